# Initial kernel scaffold; baseline (speedup 1.0000x reference)
import jax
import jax.numpy as jnp
from jax.experimental import pallas as pl


def kernel(input_tensor, check_index_tensor):
    raise NotImplementedError("write your pallas kernel here")


# speedup vs baseline: 6.8011x; 6.8011x over previous
import functools

import jax
import jax.numpy as jnp
from jax import lax
from jax.experimental import pallas as pl
from jax.experimental.pallas import tpu as pltpu
from jax.experimental.pallas import tpu_sc as plsc

_B = 32
_K = 16
_NC = 2
_NS = 16
_NW = _NC * _NS
_GROUP_M = 32
_JPG = 4
_GPW = 50
_M_PAD = _NW * _GPW * _GROUP_M
_NBUF = 2
_SIGN_MASK = jnp.int32(-(2**31))


def _sc_body(table_hbm, idx_hbm, out_hbm, idx_all, rows_v, out_v, gsem, osem):
    wid = lax.axis_index("s") * _NC + lax.axis_index("c")
    g0 = wid * _GPW
    pltpu.sync_copy(idx_hbm.at[wid], idx_all)

    def fire(slot, g):
        for j in range(_JPG):
            pltpu.async_copy(
                table_hbm.at[idx_all.at[g * _JPG + j]],
                rows_v.at[slot, pl.ds(j * 128, 128)],
                gsem,
            )

    def drain_rows(slot, g):
        for j in range(_JPG):
            pltpu.make_async_copy(
                table_hbm.at[idx_all.at[g * _JPG + j]],
                rows_v.at[slot, pl.ds(j * 128, 128)],
                gsem,
            ).wait()

    def flush_out(oslot, g):
        pltpu.async_copy(
            out_v.at[oslot],
            out_hbm.at[pl.ds((g0 + g) * _GROUP_M, _GROUP_M)],
            osem,
        )

    def drain_out(oslot):
        pltpu.make_async_copy(
            out_v.at[oslot],
            out_hbm.at[pl.ds(0, _GROUP_M)],
            osem,
        ).wait()

    def compute(slot, oslot):
        def mi_body(mi, carry):
            base = mi * _K
            for h in range(2):
                acci = jnp.zeros((16,), jnp.int32)
                accm = jnp.full((16,), 1e30, jnp.float32)
                for n in range(_K):
                    v = rows_v[slot, base + n, pl.ds(h * 16, 16)]
                    acci = acci ^ lax.bitcast_convert_type(v + 1e-10, jnp.int32)
                    av = jnp.abs(v)
                    av = jnp.where(v == 0.0, 1e10, av)
                    accm = jnp.minimum(accm, av)
                ob = lax.bitcast_convert_type(accm, jnp.int32) | (acci & _SIGN_MASK)
                out_v[oslot, mi, pl.ds(h * 16, 16)] = lax.bitcast_convert_type(
                    ob, jnp.float32
                )
            return carry

        lax.fori_loop(0, _GROUP_M, mi_body, 0)

    for b in range(_NBUF):
        fire(b, b)

    @pl.loop(0, _GPW, step=_NBUF)
    def _outer(gg):
        for b in range(_NBUF):
            g = gg + b
            drain_rows(b, g)

            @pl.when(g >= _NBUF)
            def _():
                drain_out(b)

            compute(b, b)
            flush_out(b, g)

            @pl.when(g + _NBUF < _GPW)
            def _():
                fire(b, g + _NBUF)

    for b in range(_NBUF):
        drain_out(b)


@functools.cache
def _sc_kernel():
    return pl.kernel(
        _sc_body,
        out_type=jax.ShapeDtypeStruct((_M_PAD, _B), jnp.float32),
        mesh=plsc.VectorSubcoreMesh(core_axis_name="c", subcore_axis_name="s"),
        compiler_params=pltpu.CompilerParams(use_tc_tiling_on_sc=False),
        scratch_types=[
            pltpu.VMEM((_GPW * _JPG, 128), jnp.int32),
            pltpu.VMEM((_NBUF, _JPG * 128, _B), jnp.float32),
            pltpu.VMEM((_NBUF, _GROUP_M, _B), jnp.float32),
            pltpu.SemaphoreType.DMA,
            pltpu.SemaphoreType.DMA,
        ],
    )


def kernel(input_tensor, check_index_tensor):
    batch, num_nodes = input_tensor.shape
    num_rows, _ = check_index_tensor.shape
    table = input_tensor.T
    idx = check_index_tensor.astype(jnp.int32).reshape(-1)
    pad = _M_PAD * _K - idx.shape[0]
    idx = jnp.concatenate([idx, jnp.zeros((pad,), jnp.int32)])
    idx = idx.reshape(_NW, _GPW * _JPG, 128)
    out = _sc_kernel()(table, idx)
    return out[:num_rows].T
